# baseline (speedup 1.0000x reference)
import functools

import jax
import jax.numpy as jnp
from jax import lax
from jax.experimental import pallas as pl
from jax.experimental.pallas import tpu as pltpu
from jax.experimental.pallas import tpu_sc as plsc

NUM_ENT = 1000000
EDIM = 32
HDIM = 32
B = 16384

NC = 2
NS = 16
L = 16
NW = NC * NS
BPW = B // NW
GCHUNK = 128


def _score_body(head_hbm, rel_hbm, tail_hbm, er_hbm, ei_hbm, rr_hbm, ri_hbm,
                out_hbm,
                hidx, tidx, relv, hr, hi, tr, ti, rrv, riv, sco, sem):
    wid = lax.axis_index("s") * NC + lax.axis_index("c")
    base = wid * BPW
    pltpu.sync_copy(head_hbm.at[pl.ds(base, BPW)], hidx)
    pltpu.sync_copy(tail_hbm.at[pl.ds(base, BPW)], tidx)
    pltpu.sync_copy(rel_hbm.at[pl.ds(base, BPW)], relv)
    pltpu.sync_copy(rr_hbm, rrv)
    pltpu.sync_copy(ri_hbm, riv)

    copies = []
    for c in range(BPW // GCHUNK):
        s = c * GCHUNK
        copies.append(pltpu.async_copy(
            er_hbm.at[hidx.at[pl.ds(s, GCHUNK)]], hr.at[pl.ds(s, GCHUNK)], sem))
        copies.append(pltpu.async_copy(
            ei_hbm.at[hidx.at[pl.ds(s, GCHUNK)]], hi.at[pl.ds(s, GCHUNK)], sem))
        copies.append(pltpu.async_copy(
            er_hbm.at[tidx.at[pl.ds(s, GCHUNK)]], tr.at[pl.ds(s, GCHUNK)], sem))
        copies.append(pltpu.async_copy(
            ei_hbm.at[tidx.at[pl.ds(s, GCHUNK)]], ti.at[pl.ds(s, GCHUNK)], sem))
    for cp in copies:
        cp.wait()

    rr0 = [rrv[0, pl.ds(k * L, L)] for k in range(EDIM // L)]
    ri0 = [riv[0, pl.ds(k * L, L)] for k in range(EDIM // L)]
    rr1 = [rrv[1, pl.ds(k * L, L)] for k in range(EDIM // L)]
    ri1 = [riv[1, pl.ds(k * L, L)] for k in range(EDIM // L)]
    lane = lax.broadcasted_iota(jnp.int32, (L,), 0)

    _gd = lax.GatherDimensionNumbers(
        offset_dims=(), collapsed_slice_dims=(0,), start_index_map=(0,))
    perm_idx = [(lane ^ sh)[:, None] for sh in (8, 4, 2, 1)]

    def lane_sum(v):
        for idx in perm_idx:
            v = v + lax.gather(v, idx, _gd, (1,),
                               mode=lax.GatherScatterMode.PROMISE_IN_BOUNDS)
        return v

    def group(g, carry):
        relc = relv[pl.ds(g * L, L)]
        res0 = jnp.zeros((L,), jnp.float32)
        res1 = jnp.zeros((L,), jnp.float32)
        for j in range(L):
            i = g * L + j
            s0vec = jnp.zeros((L,), jnp.float32)
            s1vec = jnp.zeros((L,), jnp.float32)
            for k in range(EDIM // L):
                hrk = hr[i, pl.ds(k * L, L)]
                hik = hi[i, pl.ds(k * L, L)]
                trk = tr[i, pl.ds(k * L, L)]
                tik = ti[i, pl.ds(k * L, L)]
                u0 = hrk * rr0[k] - hik * ri0[k]
                v0 = hik * rr0[k] + hrk * ri0[k]
                s0vec = s0vec + trk * u0 + tik * v0
                u1 = hrk * rr1[k] - hik * ri1[k]
                v1 = hik * rr1[k] + hrk * ri1[k]
                s1vec = s1vec + trk * u1 + tik * v1
            s0 = lane_sum(s0vec)
            s1 = lane_sum(s1vec)
            res0 = jnp.where(lane == j, s0, res0)
            res1 = jnp.where(lane == j, s1, res1)
        res = jnp.where(relc == 0, res0, res1)
        sco[pl.ds(g * L, L)] = res
        return carry

    lax.fori_loop(0, BPW // L, group, 0)
    pltpu.sync_copy(sco, out_hbm.at[pl.ds(base, BPW)])


def _score_sc(head, relation, tail, ent_real, ent_imag, rel_real, rel_imag):
    mesh = plsc.VectorSubcoreMesh(core_axis_name="c", subcore_axis_name="s", num_cores=NC, num_subcores=NS)
    fn = pl.kernel(
        _score_body,
        out_type=jax.ShapeDtypeStruct((B,), jnp.float32),
        mesh=mesh,
        scratch_types=[
            pltpu.VMEM((BPW,), jnp.int32),
            pltpu.VMEM((BPW,), jnp.int32),
            pltpu.VMEM((BPW,), jnp.int32),
            pltpu.VMEM((BPW, EDIM), jnp.float32),
            pltpu.VMEM((BPW, EDIM), jnp.float32),
            pltpu.VMEM((BPW, EDIM), jnp.float32),
            pltpu.VMEM((BPW, EDIM), jnp.float32),
            pltpu.VMEM((2, EDIM), jnp.float32),
            pltpu.VMEM((2, EDIM), jnp.float32),
            pltpu.VMEM((BPW,), jnp.float32),
            pltpu.SemaphoreType.DMA,
        ],
        compiler_params=pltpu.CompilerParams(use_tc_tiling_on_sc=False),
    )
    return fn(head, relation, tail, ent_real, ent_imag, rel_real, rel_imag)


RB = 10000


def _fc_body(er_ref, ei_ref, wt_ref, b_ref, out_ref):
    x = er_ref[...] + ei_ref[...]
    y = jnp.dot(x, wt_ref[...], preferred_element_type=jnp.float32)
    out_ref[...] = jnp.maximum(y + b_ref[...], 0.0)


def _node_features(ent_real, ent_imag, fc_w, fc_b):
    wt = fc_w.T
    b2 = fc_b[None, :]
    return pl.pallas_call(
        _fc_body,
        grid=(NUM_ENT // RB,),
        in_specs=[
            pl.BlockSpec((RB, EDIM), lambda i: (i, 0)),
            pl.BlockSpec((RB, EDIM), lambda i: (i, 0)),
            pl.BlockSpec((EDIM, HDIM), lambda i: (0, 0)),
            pl.BlockSpec((1, HDIM), lambda i: (0, 0)),
        ],
        out_specs=pl.BlockSpec((RB, HDIM), lambda i: (i, 0)),
        out_shape=jax.ShapeDtypeStruct((NUM_ENT, HDIM), jnp.float32),
    )(ent_real, ent_imag, wt, b2)


def kernel(head, relation, tail, edge_index, edge_type,
           ent_real, ent_imag, rel_real, rel_imag, fc_w, fc_b):
    head = head.astype(jnp.int32)
    tail = tail.astype(jnp.int32)
    relation = relation.astype(jnp.int32)
    score = _score_sc(head, relation, tail, ent_real, ent_imag,
                      rel_real, rel_imag)
    node_features = _node_features(ent_real, ent_imag, fc_w, fc_b)
    return (score, node_features)


# speedup vs baseline: 1.2939x; 1.2939x over previous
import jax
import jax.numpy as jnp
from jax import lax
from jax.experimental import pallas as pl
from jax.experimental.pallas import tpu as pltpu
from jax.experimental.pallas import tpu_sc as plsc

NUM_ENT = 1000000
EDIM = 32
HDIM = 32
B = 16384
PACK = 4
NROW = NUM_ENT // PACK
W4 = PACK * EDIM

NC = 2
NS = 16
L = 16
NW = NC * NS
BPW = B // NW
CH = 128
NCHUNK = BPW // CH


def _score_body(head_hbm, rel_hbm, tail_hbm, er_hbm, ei_hbm, rel_tab_hbm,
                out_hbm,
                hidx, tidx, relv, hgrp, tgrp, pkv, hr, hi, tr, ti, rtab, sco,
                sem):
    wid = lax.axis_index("s") * NC + lax.axis_index("c")
    base = wid * BPW
    pltpu.sync_copy(head_hbm.at[pl.ds(base, BPW)], hidx)
    pltpu.sync_copy(tail_hbm.at[pl.ds(base, BPW)], tidx)
    pltpu.sync_copy(rel_hbm.at[pl.ds(base, BPW)], relv)
    pltpu.sync_copy(rel_tab_hbm, rtab)

    def mkgrp(s, _):
        hc = hidx[pl.ds(s * L, L)]
        tc = tidx[pl.ds(s * L, L)]
        rc = relv[pl.ds(s * L, L)]
        hgrp[pl.ds(s * L, L)] = lax.shift_right_logical(hc, 2)
        tgrp[pl.ds(s * L, L)] = lax.shift_right_logical(tc, 2)
        pkv[pl.ds(s * L, L)] = ((hc & 3) | lax.shift_left(tc & 3, 2)
                                | lax.shift_left(rc, 4))
        return _
    lax.fori_loop(0, BPW // L, mkgrp, 0)

    rr0 = [rtab[pl.ds(k * L, L)] for k in range(2)]
    ri0 = [rtab[pl.ds(EDIM + k * L, L)] for k in range(2)]
    rr1 = [rtab[pl.ds(2 * EDIM + k * L, L)] for k in range(2)]
    ri1 = [rtab[pl.ds(3 * EDIM + k * L, L)] for k in range(2)]
    lane = lax.broadcasted_iota(jnp.int32, (L,), 0)

    _gd = lax.GatherDimensionNumbers(
        offset_dims=(), collapsed_slice_dims=(0,), start_index_map=(0,))

    def vperm(v, idx):
        return lax.gather(v, idx[:, None], _gd, (1,),
                          mode=lax.GatherScatterMode.PROMISE_IN_BOUNDS)

    perm_idx = [lane ^ sh for sh in (8, 4, 2, 1)]

    def lane_sum(v):
        for idx in perm_idx:
            v = v + vperm(v, idx)
        return v

    jconst = [jnp.full((L,), j, jnp.int32) for j in range(L)]

    for c in range(NCHUNK):
        s = c * CH
        cps = [
            pltpu.async_copy(er_hbm.at[hgrp.at[pl.ds(s, CH)]], hr, sem),
            pltpu.async_copy(ei_hbm.at[hgrp.at[pl.ds(s, CH)]], hi, sem),
            pltpu.async_copy(er_hbm.at[tgrp.at[pl.ds(s, CH)]], tr, sem),
            pltpu.async_copy(ei_hbm.at[tgrp.at[pl.ds(s, CH)]], ti, sem),
        ]
        for cp in cps:
            cp.wait()

        def group(g, carry):
            pkc = pkv[pl.ds(s + g * L, L)]
            res = jnp.zeros((L,), jnp.float32)
            for j in range(L):
                r0 = g * L + j
                w = pkc[j]
                hoff = (w & 3) * EDIM
                toff = ((w >> 2) & 3) * EDIM
                rsel = w >> 4
                acc = jnp.zeros((L,), jnp.float32)
                for k in range(2):
                    hrk = hr[r0, pl.ds(hoff + k * L, L)]
                    hik = hi[r0, pl.ds(hoff + k * L, L)]
                    trk = tr[r0, pl.ds(toff + k * L, L)]
                    tik = ti[r0, pl.ds(toff + k * L, L)]
                    rrk = jnp.where(rsel == 0, rr0[k], rr1[k])
                    rik = jnp.where(rsel == 0, ri0[k], ri1[k])
                    u = hrk * rrk - hik * rik
                    v = hik * rrk + hrk * rik
                    acc = acc + trk * u + tik * v
                ssum = lane_sum(acc)
                res = jnp.where(lane == j, ssum, res)
            sco[pl.ds(s + g * L, L)] = res
            return carry

        lax.fori_loop(0, CH // L, group, 0)

    pltpu.sync_copy(sco, out_hbm.at[pl.ds(base, BPW)])


def _score_sc(head, relation, tail, er4, ei4, rel_tab):
    mesh = plsc.VectorSubcoreMesh(core_axis_name="c", subcore_axis_name="s",
                                  num_cores=NC, num_subcores=NS)
    fn = pl.kernel(
        _score_body,
        out_type=jax.ShapeDtypeStruct((B,), jnp.float32),
        mesh=mesh,
        scratch_types=[
            pltpu.VMEM((BPW,), jnp.int32),
            pltpu.VMEM((BPW,), jnp.int32),
            pltpu.VMEM((BPW,), jnp.int32),
            pltpu.VMEM((BPW,), jnp.int32),
            pltpu.VMEM((BPW,), jnp.int32),
            pltpu.VMEM((BPW,), jnp.int32),
            pltpu.VMEM((CH, W4), jnp.float32),
            pltpu.VMEM((CH, W4), jnp.float32),
            pltpu.VMEM((CH, W4), jnp.float32),
            pltpu.VMEM((CH, W4), jnp.float32),
            pltpu.VMEM((4 * EDIM,), jnp.float32),
            pltpu.VMEM((BPW,), jnp.float32),
            pltpu.SemaphoreType.DMA,
        ],
    )
    return fn(head, relation, tail, er4, ei4, rel_tab)


RB4 = 5000


def _fc_body(er_ref, ei_ref, w_ref, b_ref, out_ref):
    x = er_ref[...] + ei_ref[...]
    y = jnp.dot(x, w_ref[...], preferred_element_type=jnp.float32)
    out_ref[...] = jnp.maximum(y + b_ref[...], 0.0)


def _node_features(er4, ei4, fc_w, fc_b):
    wt = fc_w.T
    wblk = jnp.zeros((W4, W4), jnp.float32)
    for p in range(PACK):
        wblk = lax.dynamic_update_slice(wblk, wt, (p * EDIM, p * HDIM))
    b4 = jnp.tile(fc_b, PACK)[None, :]
    return pl.pallas_call(
        _fc_body,
        grid=(NROW // RB4,),
        in_specs=[
            pl.BlockSpec((RB4, W4), lambda i: (i, 0)),
            pl.BlockSpec((RB4, W4), lambda i: (i, 0)),
            pl.BlockSpec((W4, W4), lambda i: (0, 0)),
            pl.BlockSpec((1, W4), lambda i: (0, 0)),
        ],
        out_specs=pl.BlockSpec((RB4, W4), lambda i: (i, 0)),
        out_shape=jax.ShapeDtypeStruct((NROW, W4), jnp.float32),
    )(er4, ei4, wblk, b4)


def kernel(head, relation, tail, edge_index, edge_type,
           ent_real, ent_imag, rel_real, rel_imag, fc_w, fc_b):
    head = head.astype(jnp.int32)
    tail = tail.astype(jnp.int32)
    relation = relation.astype(jnp.int32)
    er4 = ent_real.reshape(NROW, W4)
    ei4 = ent_imag.reshape(NROW, W4)
    rel_tab = jnp.concatenate([
        rel_real[0], rel_imag[0], rel_real[1], rel_imag[1]])
    score = _score_sc(head, relation, tail, er4, ei4, rel_tab)
    nf4 = _node_features(er4, ei4, fc_w, fc_b)
    return (score, nf4.reshape(NUM_ENT, HDIM))


# speedup vs baseline: 1.3072x; 1.0103x over previous
import jax
import jax.numpy as jnp
from jax import lax
from jax.experimental import pallas as pl
from jax.experimental.pallas import tpu as pltpu
from jax.experimental.pallas import tpu_sc as plsc

NUM_ENT = 1000000
EDIM = 32
HDIM = 32
B = 16384
PACK = 4
NROW = NUM_ENT // PACK
W4 = PACK * EDIM

NC = 2
NS = 16
L = 16
NW = NC * NS
BPW = B // NW
CH = 128
NCHUNK = BPW // CH


def _score_body(head_hbm, rel_hbm, tail_hbm, er_hbm, ei_hbm, rel_tab_hbm,
                out_hbm,
                hidx, tidx, relv, hgrp, tgrp, pkv, hr, hi, tr, ti, rtab, sco,
                sem):
    wid = lax.axis_index("s") * NC + lax.axis_index("c")
    base = wid * BPW
    pltpu.sync_copy(head_hbm.at[pl.ds(base, BPW)], hidx)
    pltpu.sync_copy(tail_hbm.at[pl.ds(base, BPW)], tidx)
    pltpu.sync_copy(rel_hbm.at[pl.ds(base, BPW)], relv)
    pltpu.sync_copy(rel_tab_hbm, rtab)

    def mkgrp(s, _):
        hc = hidx[pl.ds(s * L, L)]
        tc = tidx[pl.ds(s * L, L)]
        rc = relv[pl.ds(s * L, L)]
        one = jnp.full((L,), 1, jnp.int32)
        zero = jnp.full((L,), 0, jnp.int32)
        hp = (jnp.where(hc >= NROW, one, zero)
              + jnp.where(hc >= 2 * NROW, one, zero)
              + jnp.where(hc >= 3 * NROW, one, zero))
        tp = (jnp.where(tc >= NROW, one, zero)
              + jnp.where(tc >= 2 * NROW, one, zero)
              + jnp.where(tc >= 3 * NROW, one, zero))
        hgrp[pl.ds(s * L, L)] = hc - hp * NROW
        tgrp[pl.ds(s * L, L)] = tc - tp * NROW
        pkv[pl.ds(s * L, L)] = (hp | lax.shift_left(tp, 2)
                                | lax.shift_left(rc, 4))
        return _
    lax.fori_loop(0, BPW // L, mkgrp, 0)

    rr0 = [rtab[pl.ds(k * L, L)] for k in range(2)]
    ri0 = [rtab[pl.ds(EDIM + k * L, L)] for k in range(2)]
    rr1 = [rtab[pl.ds(2 * EDIM + k * L, L)] for k in range(2)]
    ri1 = [rtab[pl.ds(3 * EDIM + k * L, L)] for k in range(2)]
    lane = lax.broadcasted_iota(jnp.int32, (L,), 0)

    _gd = lax.GatherDimensionNumbers(
        offset_dims=(), collapsed_slice_dims=(0,), start_index_map=(0,))

    def vperm(v, idx):
        return lax.gather(v, idx[:, None], _gd, (1,),
                          mode=lax.GatherScatterMode.PROMISE_IN_BOUNDS)

    perm_idx = [lane ^ sh for sh in (8, 4, 2, 1)]

    def lane_sum(v):
        for idx in perm_idx:
            v = v + vperm(v, idx)
        return v

    jconst = [jnp.full((L,), j, jnp.int32) for j in range(L)]

    for c in range(NCHUNK):
        s = c * CH
        cps = [
            pltpu.async_copy(er_hbm.at[hgrp.at[pl.ds(s, CH)]], hr, sem),
            pltpu.async_copy(ei_hbm.at[hgrp.at[pl.ds(s, CH)]], hi, sem),
            pltpu.async_copy(er_hbm.at[tgrp.at[pl.ds(s, CH)]], tr, sem),
            pltpu.async_copy(ei_hbm.at[tgrp.at[pl.ds(s, CH)]], ti, sem),
        ]
        for cp in cps:
            cp.wait()

        def group(g, carry):
            pkc = pkv[pl.ds(s + g * L, L)]
            res = jnp.zeros((L,), jnp.float32)
            for j in range(L):
                r0 = g * L + j
                w = pkc[j]
                hoff = (w & 3) * EDIM
                toff = ((w >> 2) & 3) * EDIM
                rsel = w >> 4
                acc = jnp.zeros((L,), jnp.float32)
                for k in range(2):
                    hrk = hr[r0, pl.ds(hoff + k * L, L)]
                    hik = hi[r0, pl.ds(hoff + k * L, L)]
                    trk = tr[r0, pl.ds(toff + k * L, L)]
                    tik = ti[r0, pl.ds(toff + k * L, L)]
                    rrk = jnp.where(rsel == 0, rr0[k], rr1[k])
                    rik = jnp.where(rsel == 0, ri0[k], ri1[k])
                    u = hrk * rrk - hik * rik
                    v = hik * rrk + hrk * rik
                    acc = acc + trk * u + tik * v
                ssum = lane_sum(acc)
                res = jnp.where(lane == j, ssum, res)
            sco[pl.ds(s + g * L, L)] = res
            return carry

        lax.fori_loop(0, CH // L, group, 0)

    pltpu.sync_copy(sco, out_hbm.at[pl.ds(base, BPW)])


def _score_sc(head, relation, tail, er4, ei4, rel_tab):
    mesh = plsc.VectorSubcoreMesh(core_axis_name="c", subcore_axis_name="s",
                                  num_cores=NC, num_subcores=NS)
    fn = pl.kernel(
        _score_body,
        out_type=jax.ShapeDtypeStruct((B,), jnp.float32),
        mesh=mesh,
        scratch_types=[
            pltpu.VMEM((BPW,), jnp.int32),
            pltpu.VMEM((BPW,), jnp.int32),
            pltpu.VMEM((BPW,), jnp.int32),
            pltpu.VMEM((BPW,), jnp.int32),
            pltpu.VMEM((BPW,), jnp.int32),
            pltpu.VMEM((BPW,), jnp.int32),
            pltpu.VMEM((CH, W4), jnp.float32),
            pltpu.VMEM((CH, W4), jnp.float32),
            pltpu.VMEM((CH, W4), jnp.float32),
            pltpu.VMEM((CH, W4), jnp.float32),
            pltpu.VMEM((4 * EDIM,), jnp.float32),
            pltpu.VMEM((BPW,), jnp.float32),
            pltpu.SemaphoreType.DMA,
        ],
    )
    return fn(head, relation, tail, er4, ei4, rel_tab)


RBQ = 5000
NBQ = NROW // RBQ


def _fc_body(er_ref, ei_ref, wt_ref, b_ref, nf_ref, er4_ref, ei4_ref):
    p = pl.program_id(1)
    er = er_ref[...]
    ei = ei_ref[...]
    x = er + ei
    y = jnp.dot(x, wt_ref[...], preferred_element_type=jnp.float32)
    nf_ref[...] = jnp.maximum(y + b_ref[...], 0.0)
    for q in range(PACK):
        @pl.when(p == q)
        def _():
            er4_ref[:, q * EDIM:(q + 1) * EDIM] = er
            ei4_ref[:, q * EDIM:(q + 1) * EDIM] = ei


def _node_features_and_pack(ent_real, ent_imag, fc_w, fc_b):
    wt = fc_w.T
    b2 = fc_b[None, :]
    return pl.pallas_call(
        _fc_body,
        grid=(NBQ, PACK),
        in_specs=[
            pl.BlockSpec((RBQ, EDIM), lambda i, p: (p * NBQ + i, 0)),
            pl.BlockSpec((RBQ, EDIM), lambda i, p: (p * NBQ + i, 0)),
            pl.BlockSpec((EDIM, HDIM), lambda i, p: (0, 0)),
            pl.BlockSpec((1, HDIM), lambda i, p: (0, 0)),
        ],
        out_specs=[
            pl.BlockSpec((RBQ, HDIM), lambda i, p: (p * NBQ + i, 0)),
            pl.BlockSpec((RBQ, W4), lambda i, p: (i, 0)),
            pl.BlockSpec((RBQ, W4), lambda i, p: (i, 0)),
        ],
        out_shape=[
            jax.ShapeDtypeStruct((NUM_ENT, HDIM), jnp.float32),
            jax.ShapeDtypeStruct((NROW, W4), jnp.float32),
            jax.ShapeDtypeStruct((NROW, W4), jnp.float32),
        ],
    )(ent_real, ent_imag, wt, b2)


def kernel(head, relation, tail, edge_index, edge_type,
           ent_real, ent_imag, rel_real, rel_imag, fc_w, fc_b):
    head = head.astype(jnp.int32)
    tail = tail.astype(jnp.int32)
    relation = relation.astype(jnp.int32)
    rel_tab = jnp.concatenate([
        rel_real[0], rel_imag[0], rel_real[1], rel_imag[1]])
    nf, er4, ei4 = _node_features_and_pack(ent_real, ent_imag, fc_w, fc_b)
    score = _score_sc(head, relation, tail, er4, ei4, rel_tab)
    return (score, nf)
